# baseline (speedup 1.0000x reference)
import functools

import jax
import jax.numpy as jnp
from jax import lax
from jax.experimental import pallas as pl
from jax.experimental.pallas import tpu as pltpu
from jax.experimental.pallas import tpu_sc as plsc

N = 10000
E = 320000
D = 128
B = 64
NP_PAD = 10240
M = 2 * NP_PAD
NC, NS, LANES = 2, 16, 16
K = 128
NBUF = 7
ESL = E + NP_PAD
CH = NBUF * (-(-ESL // (NS * K * NBUF)))
EEPG = CH * NS * K
PAD_ROW = NP_PAD - 1
NPS = NP_PAD // NS
NGRP = NPS // LANES
DEG_ROUND = 14
RB = 256
NBLK = M // RB
NBP = NP_PAD // RB
F32 = jnp.float32

_MESH = dict(core_axis_name="c", subcore_axis_name="s",
             num_cores=NC, num_subcores=NS)
def _sc_params():
    return dict(
        mesh=plsc.VectorSubcoreMesh(**_MESH),
        compiler_params=pltpu.CompilerParams(use_tc_tiling_on_sc=False,
                                             needs_layout_passes=False),
    )


def _dot(a, b):
    return lax.dot(a, b, precision=lax.Precision.HIGHEST,
                   preferred_element_type=F32)


def _dot_t(a, b):
    return lax.dot_general(a, b, (((0,), (0,)), ((), ())),
                           precision=lax.Precision.HIGHEST,
                           preferred_element_type=F32)


def _splat(v):
    return jnp.full((LANES,), v, jnp.int32)


def _fill_rows(ref, nrows, ncols, value):
    vec = jnp.full((LANES,), value, F32)
    for r in range(nrows):
        for c in range(ncols // LANES):
            ref[r, pl.ds(c * LANES, LANES)] = vec


def _zero_slice(zsrc128, acc, base):
    for t in range(NPS // 128):
        pltpu.sync_copy(zsrc128, acc.at[pl.ds(base + t * 128, 128)])


@functools.lru_cache(maxsize=None)
def _get_sc_layers():
    @functools.partial(
        pl.kernel,
        out_type=[
            jax.ShapeDtypeStruct((M, 16), F32),
            jax.ShapeDtypeStruct((M, 16), F32),
            jax.ShapeDtypeStruct((M, 16), F32),
            jax.ShapeDtypeStruct((M, 32), F32),
        ],
        scratch_types=[
            pltpu.VMEM((CH, K), jnp.int32),
            pltpu.VMEM((CH, K), jnp.int32),
            pltpu.VMEM((NBUF * K, 32), F32),
            pltpu.VMEM((NBUF * K, 16), F32),
            pltpu.VMEM((NPS, 16), F32),
            pltpu.VMEM((NPS,), F32),
            pltpu.VMEM((32, 16), F32),
            pltpu.VMEM((16, 16), F32),
            pltpu.VMEM((2, 16), F32),
            pltpu.VMEM((1, 16), F32),
            pltpu.VMEM((1, 16), F32),
            pltpu.VMEM_SHARED((NP_PAD, 32), F32),
            pltpu.VMEM_SHARED((NP_PAD, 16), F32),
            pltpu.SemaphoreType.DMA,
            pltpu.SemaphoreType.DMA,
        ],
        **_sc_params(),
    )
    def layers(y1_hbm, src_hbm, dst_hbm, w2_hbm, w3_hbm,
               b1_hbm, b2_hbm, b3_hbm,
               h3_hbm, z2_hbm, z3_hbm, z1_hbm,
               src_v, dst_v, gbuf32, buf16, bufa16, dbuf,
               w2v, w3v, b1v, b2v, b3v, acc32, acc16, gsem, ssem):
        ci = lax.axis_index("c")
        si = lax.axis_index("s")
        lo_base = si * NPS
        gl_base = ci * NP_PAD + lo_base

        pltpu.sync_copy(src_hbm.at[ci, si], src_v)
        pltpu.sync_copy(dst_hbm.at[ci, si], dst_v)
        pltpu.sync_copy(w2_hbm.at[ci], w2v)
        pltpu.sync_copy(w3_hbm.at[ci], w3v)
        pltpu.sync_copy(b1_hbm.at[ci], b1v)
        pltpu.sync_copy(b2_hbm.at[ci], b2v)
        pltpu.sync_copy(b3_hbm.at[ci], b3v)
        _fill_rows(gbuf32, 128, 32, 0.0)
        _fill_rows(bufa16, 128, 16, 0.0)
        _fill_rows(buf16, K, 16, 1.0)
        _zero_slice(gbuf32.at[pl.ds(0, 128)], acc32, lo_base)
        _zero_slice(bufa16.at[pl.ds(0, 128)], acc16, lo_base)
        plsc.subcore_barrier()

        def deg_round(o, carry):
            descs = []
            for b in range(DEG_ROUND):
                descs.append(pltpu.async_copy(
                    buf16.at[pl.ds(0, K)],
                    acc16.at[dst_v.at[o * DEG_ROUND + b]], ssem, add=True))
            for d in descs:
                d.wait()
            return carry

        lax.fori_loop(0, CH // DEG_ROUND, deg_round, 0)
        plsc.subcore_barrier()

        pltpu.sync_copy(acc16.at[pl.ds(lo_base, NPS)], bufa16)

        def dinv_grp(g, carry):
            base = g * LANES
            rows = base + lax.iota(jnp.int32, LANES)
            degv = plsc.load_gather(bufa16, [rows, _splat(0)])
            xv = jnp.maximum(degv, 1.0)
            yi = jnp.int32(0x5F3759DF) - (plsc.bitcast(xv, jnp.int32) >> 1)
            y = plsc.bitcast(yi, F32)
            for _ in range(3):
                y = y * (1.5 - 0.5 * xv * y * y)
            dbuf[pl.ds(base, LANES)] = y
            return carry

        lax.fori_loop(0, NGRP, dinv_grp, 0)
        _fill_rows(bufa16, 128, 16, 0.0)
        _zero_slice(bufa16.at[pl.ds(0, 128)], acc16, lo_base)

        pltpu.sync_copy(y1_hbm.at[pl.ds(gl_base, NPS)],
                        gbuf32.at[pl.ds(0, NPS)])

        def scale_grp(g, carry):
            base = g * LANES
            rows = base + lax.iota(jnp.int32, LANES)
            dv = dbuf[pl.ds(base, LANES)]
            for k in range(32):
                colk = plsc.load_gather(gbuf32, [rows, _splat(k)])
                plsc.store_scatter(gbuf32, [rows, _splat(k)], colk * dv)
            return carry

        lax.fori_loop(0, NGRP, scale_grp, 0)
        pltpu.sync_copy(gbuf32.at[pl.ds(0, NPS)],
                        z1_hbm.at[pl.ds(gl_base, NPS)])
        plsc.subcore_barrier()

        def do_agg(table_hbm, acc, gb):
            def drain_scatters(o):
                for b in range(NBUF):
                    pltpu.make_async_copy(
                        gb.at[pl.ds(b * K, K)],
                        acc.at[dst_v.at[o * NBUF + b]], ssem).wait()

            def rnd(o, carry):
                @pl.when(o > 0)
                def _():
                    drain_scatters(o - 1)
                gd = []
                for b in range(NBUF):
                    gd.append(pltpu.async_copy(
                        table_hbm.at[src_v.at[o * NBUF + b]],
                        gb.at[pl.ds(b * K, K)], gsem))
                for d in gd:
                    d.wait()
                for b in range(NBUF):
                    pltpu.async_copy(
                        gb.at[pl.ds(b * K, K)],
                        acc.at[dst_v.at[o * NBUF + b]], ssem, add=True)
                return carry

            nround = CH // NBUF
            lax.fori_loop(0, nround, rnd, 0)
            drain_scatters(nround - 1)

        def dense(src_vmem, fin, wv, bv, out_vmem):
            def grp(g, carry):
                base = g * LANES
                rows = base + lax.iota(jnp.int32, LANES)
                dv = dbuf[pl.ds(base, LANES)]
                brows = [bv[r, :] for r in range(fin // LANES)]
                if wv is not None:
                    wrows = [wv[k, :] for k in range(fin)]
                s = []
                for k in range(fin):
                    colk = plsc.load_gather(src_vmem, [rows, _splat(k)])
                    bk = brows[k // LANES][k % LANES]
                    sk = jnp.maximum(dv * colk + bk, 0.0)
                    if wv is None:
                        plsc.store_scatter(out_vmem, [rows, _splat(k)], sk)
                    else:
                        s.append(sk * dv)
                if wv is not None:
                    for j in range(16):
                        o = s[0] * wrows[0][j]
                        for k in range(1, fin):
                            o = o + s[k] * wrows[k][j]
                        plsc.store_scatter(out_vmem, [rows, _splat(j)], o)
                return carry
            lax.fori_loop(0, NGRP, grp, 0)

        do_agg(z1_hbm, acc32, gbuf32)
        plsc.subcore_barrier()
        pltpu.sync_copy(acc32.at[pl.ds(lo_base, NPS)],
                        gbuf32.at[pl.ds(0, NPS)])
        dense(gbuf32, 32, w2v, b1v, buf16)
        pltpu.sync_copy(buf16.at[pl.ds(0, NPS)],
                        z2_hbm.at[pl.ds(gl_base, NPS)])
        plsc.subcore_barrier()

        do_agg(z2_hbm, acc16, buf16)
        plsc.subcore_barrier()
        pltpu.sync_copy(acc16.at[pl.ds(lo_base, NPS)],
                        bufa16.at[pl.ds(0, NPS)])
        dense(bufa16, 16, w3v, b2v, buf16)
        pltpu.sync_copy(buf16.at[pl.ds(0, NPS)],
                        z3_hbm.at[pl.ds(gl_base, NPS)])
        _fill_rows(bufa16, 128, 16, 0.0)
        _zero_slice(bufa16.at[pl.ds(0, 128)], acc16, lo_base)
        plsc.subcore_barrier()

        do_agg(z3_hbm, acc16, buf16)
        plsc.subcore_barrier()
        pltpu.sync_copy(acc16.at[pl.ds(lo_base, NPS)],
                        bufa16.at[pl.ds(0, NPS)])
        dense(bufa16, 16, None, b3v, buf16)
        pltpu.sync_copy(buf16.at[pl.ds(0, NPS)],
                        h3_hbm.at[pl.ds(gl_base, NPS)])

    return layers


def _k1_body(x_ref, wp1, wd1, ga1w, gb1w, ga1b, gb1b,
             ga2w, gb2w, ga2b, gb2b,
             hw_ref, g_ref, gmp_ref, gmd_ref):
    i = pl.program_id(0)
    is_p = i < NBP
    xb = x_ref[...]
    w1 = jnp.where(is_p, wp1[...], wd1[...])
    hw_ref[...] = _dot(xb, w1)
    g1w = jnp.where(is_p, ga1w[...], gb1w[...])
    g1b = jnp.where(is_p, ga1b[...], gb1b[...])
    g2w = jnp.where(is_p, ga2w[...], gb2w[...])
    g2b = jnp.where(is_p, ga2b[...], gb2b[...])
    t = jnp.maximum(_dot(xb, g1w) + g1b, 0.0)
    g = _dot(t, g2w) + g2b
    g_ref[...] = g
    bm = jnp.max(g, keepdims=True)

    @pl.when(i == 0)
    def _():
        gmp_ref[...] = bm
        gmd_ref[...] = bm - 1.0

    @pl.when((i > 0) & is_p)
    def _():
        gmp_ref[...] = jnp.maximum(gmp_ref[...], bm)

    @pl.when(i == NBP)
    def _():
        gmd_ref[...] = bm

    @pl.when(i > NBP)
    def _():
        gmd_ref[...] = jnp.maximum(gmd_ref[...], bm)


def _tc_k1(x, p):
    return pl.pallas_call(
        _k1_body,
        grid=(NBLK,),
        in_specs=[
            pl.BlockSpec((RB, D), lambda i: (i, 0)),
        ] + [pl.BlockSpec(w.shape, lambda i: (0, 0))
             for w in (p["Wp1"], p["Wd1"], p["Ga1"], p["Gb1"],
                       p["ga1r"], p["gb1r"], p["Ga2"], p["Gb2"],
                       p["ga2r"], p["gb2r"])],
        out_specs=[
            pl.BlockSpec((RB, 32), lambda i: (i, 0)),
            pl.BlockSpec((RB, 1), lambda i: (i, 0)),
            pl.BlockSpec((1, 1), lambda i: (0, 0)),
            pl.BlockSpec((1, 1), lambda i: (0, 0)),
        ],
        out_shape=[
            jax.ShapeDtypeStruct((M, 32), F32),
            jax.ShapeDtypeStruct((M, 1), F32),
            jax.ShapeDtypeStruct((1, 1), F32),
            jax.ShapeDtypeStruct((1, 1), F32),
        ],
    )(x, p["Wp1"], p["Wd1"], p["Ga1"], p["Gb1"], p["ga1r"],
      p["gb1r"], p["Ga2"], p["Gb2"], p["ga2r"], p["gb2r"])


def _pool_body(h3_ref, g_ref, gmp_ref, gmd_ref, x_ref, batch_ref,
               l1w_ref, l1b_ref, l2w_ref, l2b_ref,
               shp, sdp, scp, sxp, shd, sdd, scd, sxd, out_ref):
    i = pl.program_id(0)
    is_p = i < NBP
    h3 = h3_ref[...]
    gm = jnp.where(is_p, gmp_ref[...], gmd_ref[...])
    ge = jnp.exp(g_ref[...] - gm)
    iota = lax.broadcasted_iota(jnp.int32, (1, B), 1).astype(F32)
    oh = (batch_ref[...] == iota).astype(F32)
    sh = _dot_t(oh, h3)
    sden = _dot_t(oh, ge)
    scnt = _dot_t(oh, jnp.ones((RB, 1), F32))
    sx = _dot_t(oh, ge * x_ref[...])

    @pl.when(i == 0)
    def _():
        shp[...] = sh
        sdp[...] = sden
        scp[...] = scnt
        sxp[...] = sx
        out_ref[...] = jnp.zeros((B, 1), F32)

    @pl.when((i > 0) & is_p)
    def _():
        shp[...] += sh
        sdp[...] += sden
        scp[...] += scnt
        sxp[...] += sx

    @pl.when(i == NBP)
    def _():
        shd[...] = sh
        sdd[...] = sden
        scd[...] = scnt
        sxd[...] = sx

    @pl.when(i > NBP)
    def _():
        shd[...] += sh
        sdd[...] += sden
        scd[...] += scnt
        sxd[...] += sx

    @pl.when(i == NBLK - 1)
    def _():
        pp = shp[...] / jnp.maximum(scp[...], 1.0)
        pd = shd[...] / jnp.maximum(scd[...], 1.0)
        ap = sxp[...] / jnp.maximum(sdp[...], 1e-12)
        ad = sxd[...] / jnp.maximum(sdd[...], 1e-12)
        z = (_dot(pp, l1w_ref[0:16]) + _dot(pd, l1w_ref[16:32])
             + _dot(ap, l1w_ref[32:160]) + _dot(ad, l1w_ref[160:288])
             + l1b_ref[...])
        z = jnp.maximum(z, 0.0)
        out_ref[...] = _dot(z, l2w_ref[...]) + l2b_ref[...]


def _tc_pool(h3, g, gmp, gmd, x, batch, l1w, l1b, l2w, l2b):
    zspec = pl.BlockSpec((1, 1), lambda i: (0, 0))
    outs = [
        pl.BlockSpec((B, 16), lambda i: (0, 0)),
        pl.BlockSpec((B, 1), lambda i: (0, 0)),
        pl.BlockSpec((B, 1), lambda i: (0, 0)),
        pl.BlockSpec((B, D), lambda i: (0, 0)),
    ]
    shapes = [
        jax.ShapeDtypeStruct((B, 16), F32),
        jax.ShapeDtypeStruct((B, 1), F32),
        jax.ShapeDtypeStruct((B, 1), F32),
        jax.ShapeDtypeStruct((B, D), F32),
    ]
    res = pl.pallas_call(
        _pool_body,
        grid=(NBLK,),
        in_specs=[
            pl.BlockSpec((RB, 16), lambda i: (i, 0)),
            pl.BlockSpec((RB, 1), lambda i: (i, 0)),
            zspec,
            zspec,
            pl.BlockSpec((RB, D), lambda i: (i, 0)),
            pl.BlockSpec((RB, 1), lambda i: (i, 0)),
            pl.BlockSpec((288, 16), lambda i: (0, 0)),
            pl.BlockSpec((1, 16), lambda i: (0, 0)),
            pl.BlockSpec((16, 1), lambda i: (0, 0)),
            pl.BlockSpec((1, 1), lambda i: (0, 0)),
        ],
        out_specs=outs + outs + [pl.BlockSpec((B, 1), lambda i: (0, 0))],
        out_shape=shapes + shapes + [jax.ShapeDtypeStruct((B, 1), F32)],
    )(h3, g, gmp, gmd, x, batch, l1w, l1b, l2w, l2b)
    return res[-1]


def kernel(x_p, x_d, edge_attr_p, edge_attr_d, edge_index_p, edge_index_d,
           batch_p, batch_d, params):
    del edge_attr_p, edge_attr_d
    p = dict(params)
    for k in ("ga1", "gb1", "ga2", "gb2", "bp1", "bp2", "bp3",
              "bd1", "bd2", "bd3", "l1", "l2"):
        p[k + "r"] = p[k].reshape(1, -1).astype(F32)

    pad_n = NP_PAD - N
    zrows = jnp.zeros((pad_n, D), F32)
    x = jnp.concatenate([x_p, zrows, x_d, zrows], axis=0)

    ar = jnp.arange(NP_PAD, dtype=jnp.int32)
    epad_n = EEPG - (E + NP_PAD)

    def edges_for(g, ei):
        srcg = jnp.concatenate([
            ei[0].astype(jnp.int32), ar,
            jnp.full((epad_n,), PAD_ROW, jnp.int32)]) + g * NP_PAD
        dstl = jnp.concatenate([
            ei[1].astype(jnp.int32), ar,
            jnp.full((epad_n,), PAD_ROW, jnp.int32)])
        return srcg.reshape(NS, CH, K), dstl.reshape(NS, CH, K)

    sp, dp_ = edges_for(0, edge_index_p)
    sd, dd_ = edges_for(1, edge_index_d)
    src4 = jnp.stack([sp, sd])
    dst4 = jnp.stack([dp_, dd_])

    bpad = jnp.full((pad_n,), B, batch_p.dtype)
    batch = jnp.concatenate([batch_p, bpad, batch_d, bpad])
    batch = batch.astype(F32).reshape(M, 1)

    y1, g, gmp, gmd = _tc_k1(x, p)
    w2 = jnp.stack([p["Wp2"], p["Wd2"]])
    w3 = jnp.stack([p["Wp3"], p["Wd3"]])
    h3, _, _, _ = _get_sc_layers()(
        y1, src4, dst4, w2, w3,
        jnp.stack([p["bp1"].reshape(2, LANES),
                   p["bd1"].reshape(2, LANES)]).astype(F32),
        jnp.stack([p["bp2r"], p["bd2r"]]),
        jnp.stack([p["bp3r"], p["bd3r"]]))
    return _tc_pool(h3, g, gmp, gmd, x, batch,
                    p["L1"], p["l1r"], p["L2"], p["l2r"])


# speedup vs baseline: 1.6599x; 1.6599x over previous
import functools

import jax
import jax.numpy as jnp
from jax import lax
from jax.experimental import pallas as pl
from jax.experimental.pallas import tpu as pltpu
from jax.experimental.pallas import tpu_sc as plsc

N = 10000
E = 320000
D = 128
B = 64
NP_PAD = 10240
M = 2 * NP_PAD
NC, NS, LANES = 2, 16, 16
K = 128
NBUF = 6
ESL = E + NP_PAD
CH = NBUF * (-(-ESL // (NS * K * NBUF)))
EEPG = CH * NS * K
PAD_ROW = NP_PAD - 1
NPS = NP_PAD // NS
NGRP = NPS // LANES
DEG_ROUND = 14
RB = 256
NBLK = M // RB
NBP = NP_PAD // RB
F32 = jnp.float32

_MESH = dict(core_axis_name="c", subcore_axis_name="s",
             num_cores=NC, num_subcores=NS)
def _sc_params():
    return dict(
        mesh=plsc.VectorSubcoreMesh(**_MESH),
        compiler_params=pltpu.CompilerParams(use_tc_tiling_on_sc=False,
                                             needs_layout_passes=False),
    )


def _dot(a, b):
    return lax.dot(a, b, precision=lax.Precision.HIGHEST,
                   preferred_element_type=F32)


def _dot_t(a, b):
    return lax.dot_general(a, b, (((0,), (0,)), ((), ())),
                           precision=lax.Precision.HIGHEST,
                           preferred_element_type=F32)


def _splat(v):
    return jnp.full((LANES,), v, jnp.int32)


def _fill_rows(ref, nrows, ncols, value):
    vec = jnp.full((LANES,), value, F32)
    for r in range(nrows):
        for c in range(ncols // LANES):
            ref[r, pl.ds(c * LANES, LANES)] = vec


def _zero_slice(zsrc128, acc, base):
    for t in range(NPS // 128):
        pltpu.sync_copy(zsrc128, acc.at[pl.ds(base + t * 128, 128)])


@functools.lru_cache(maxsize=None)
def _get_sc_degree():
    @functools.partial(
        pl.kernel,
        out_type=jax.ShapeDtypeStruct((M, LANES), F32),
        scratch_types=[
            pltpu.VMEM((CH, K), jnp.int32),
            pltpu.VMEM((K, LANES), F32),
            pltpu.VMEM((K, LANES), F32),
            pltpu.VMEM_SHARED((NP_PAD, LANES), F32),
            pltpu.SemaphoreType.DMA,
        ],
        **_sc_params(),
    )
    def deg(dst_hbm, out_hbm, dst_v, ones_v, zeros_v, acc, ssem):
        ci = lax.axis_index("c")
        si = lax.axis_index("s")
        lo_base = si * NPS
        gl_base = ci * NP_PAD + lo_base
        _fill_rows(ones_v, K, LANES, 1.0)
        _fill_rows(zeros_v, K, LANES, 0.0)
        _zero_slice(zeros_v.at[pl.ds(0, 128)], acc, lo_base)
        pltpu.sync_copy(dst_hbm.at[ci, si], dst_v)
        plsc.subcore_barrier()

        def body(o, carry):
            descs = []
            for b in range(DEG_ROUND):
                j = o * DEG_ROUND + b
                descs.append(pltpu.async_copy(
                    ones_v, acc.at[dst_v.at[j]], ssem, add=True))
            for d in descs:
                d.wait()
            return carry

        lax.fori_loop(0, CH // DEG_ROUND, body, 0)
        plsc.subcore_barrier()
        pltpu.sync_copy(acc.at[pl.ds(lo_base, NPS)],
                        out_hbm.at[pl.ds(gl_base, NPS)])

    return deg


@functools.lru_cache(maxsize=None)
def _get_sc_layers():
    @functools.partial(
        pl.kernel,
        out_type=[
            jax.ShapeDtypeStruct((M, 16), F32),
            jax.ShapeDtypeStruct((M, 16), F32),
            jax.ShapeDtypeStruct((M, 16), F32),
        ],
        scratch_types=[
            pltpu.VMEM((CH, K), jnp.int32),
            pltpu.VMEM((CH, K), jnp.int32),
            pltpu.VMEM((NBUF * K, 32), F32),
            pltpu.VMEM((NBUF * K, 16), F32),
            pltpu.VMEM((NBUF * K, 16), F32),
            pltpu.VMEM((NPS,), F32),
            pltpu.VMEM((32, 16), F32),
            pltpu.VMEM((16, 16), F32),
            pltpu.VMEM((2, 16), F32),
            pltpu.VMEM((1, 16), F32),
            pltpu.VMEM((1, 16), F32),
            pltpu.VMEM_SHARED((NP_PAD, 32), F32),
            pltpu.VMEM_SHARED((NP_PAD, 16), F32),
            pltpu.SemaphoreType.DMA,
            pltpu.SemaphoreType.DMA,
        ],
        **_sc_params(),
    )
    def layers(z1_hbm, src_hbm, dst_hbm, dinv_hbm, w2_hbm, w3_hbm,
               b1_hbm, b2_hbm, b3_hbm,
               h3_hbm, z2_hbm, z3_hbm,
               src_v, dst_v, gbuf32, buf16, bufa16, dbuf,
               w2v, w3v, b1v, b2v, b3v, acc32, acc16, gsem, ssem):
        ci = lax.axis_index("c")
        si = lax.axis_index("s")
        lo_base = si * NPS
        gl_base = ci * NP_PAD + lo_base

        pltpu.sync_copy(src_hbm.at[ci, si], src_v)
        pltpu.sync_copy(dst_hbm.at[ci, si], dst_v)
        pltpu.sync_copy(w2_hbm.at[ci], w2v)
        pltpu.sync_copy(w3_hbm.at[ci], w3v)
        pltpu.sync_copy(b1_hbm.at[ci], b1v)
        pltpu.sync_copy(b2_hbm.at[ci], b2v)
        pltpu.sync_copy(b3_hbm.at[ci], b3v)
        pltpu.sync_copy(dinv_hbm.at[pl.ds(gl_base, NPS)], dbuf)
        _fill_rows(gbuf32, 128, 32, 0.0)
        _fill_rows(bufa16, 128, 16, 0.0)
        _zero_slice(gbuf32.at[pl.ds(0, 128)], acc32, lo_base)
        _zero_slice(bufa16.at[pl.ds(0, 128)], acc16, lo_base)
        plsc.subcore_barrier()

        def do_agg(table_hbm, acc, gb):
            def drain_scatters(o):
                for b in range(NBUF):
                    pltpu.make_async_copy(
                        gb.at[pl.ds(b * K, K)],
                        acc.at[dst_v.at[o * NBUF + b]], ssem).wait()

            def rnd(o, carry):
                @pl.when(o > 0)
                def _():
                    drain_scatters(o - 1)
                gd = []
                for b in range(NBUF):
                    gd.append(pltpu.async_copy(
                        table_hbm.at[src_v.at[o * NBUF + b]],
                        gb.at[pl.ds(b * K, K)], gsem))
                for d in gd:
                    d.wait()
                for b in range(NBUF):
                    pltpu.async_copy(
                        gb.at[pl.ds(b * K, K)],
                        acc.at[dst_v.at[o * NBUF + b]], ssem, add=True)
                return carry

            nround = CH // NBUF
            lax.fori_loop(0, nround, rnd, 0)
            drain_scatters(nround - 1)

        def dense(src_vmem, fin, wv, bv, out_vmem):
            def grp(g, carry):
                base = g * LANES
                rows = base + lax.iota(jnp.int32, LANES)
                dv = dbuf[pl.ds(base, LANES)]
                brows = [bv[r, :] for r in range(fin // LANES)]
                if wv is not None:
                    wrows = [wv[k, :] for k in range(fin)]
                s = []
                for k in range(fin):
                    colk = plsc.load_gather(src_vmem, [rows, _splat(k)])
                    bk = brows[k // LANES][k % LANES]
                    sk = jnp.maximum(dv * colk + bk, 0.0)
                    if wv is None:
                        plsc.store_scatter(out_vmem, [rows, _splat(k)], sk)
                    else:
                        s.append(sk * dv)
                if wv is not None:
                    for j in range(16):
                        o = s[0] * wrows[0][j]
                        for k in range(1, fin):
                            o = o + s[k] * wrows[k][j]
                        plsc.store_scatter(out_vmem, [rows, _splat(j)], o)
                return carry
            lax.fori_loop(0, NGRP, grp, 0)

        do_agg(z1_hbm, acc32, gbuf32)
        plsc.subcore_barrier()
        pltpu.sync_copy(acc32.at[pl.ds(lo_base, NPS)],
                        gbuf32.at[pl.ds(0, NPS)])
        dense(gbuf32, 32, w2v, b1v, buf16)
        pltpu.sync_copy(buf16.at[pl.ds(0, NPS)],
                        z2_hbm.at[pl.ds(gl_base, NPS)])
        plsc.subcore_barrier()

        do_agg(z2_hbm, acc16, buf16)
        plsc.subcore_barrier()
        pltpu.sync_copy(acc16.at[pl.ds(lo_base, NPS)],
                        bufa16.at[pl.ds(0, NPS)])
        dense(bufa16, 16, w3v, b2v, buf16)
        pltpu.sync_copy(buf16.at[pl.ds(0, NPS)],
                        z3_hbm.at[pl.ds(gl_base, NPS)])
        _fill_rows(bufa16, 128, 16, 0.0)
        _zero_slice(bufa16.at[pl.ds(0, 128)], acc16, lo_base)
        plsc.subcore_barrier()

        do_agg(z3_hbm, acc16, buf16)
        plsc.subcore_barrier()
        pltpu.sync_copy(acc16.at[pl.ds(lo_base, NPS)],
                        bufa16.at[pl.ds(0, NPS)])
        dense(bufa16, 16, None, b3v, buf16)
        pltpu.sync_copy(buf16.at[pl.ds(0, NPS)],
                        h3_hbm.at[pl.ds(gl_base, NPS)])

    return layers


def _k1_body(x_ref, dacc_ref, wp1, wd1, ga1w, gb1w, ga1b, gb1b,
             ga2w, gb2w, ga2b, gb2b,
             hw_ref, dinv_ref, g_ref, gmp_ref, gmd_ref):
    i = pl.program_id(0)
    is_p = i < NBP
    deg = dacc_ref[:, 0:1]
    dinv = lax.rsqrt(jnp.maximum(deg, 1.0))
    xb = x_ref[...]
    w1 = jnp.where(is_p, wp1[...], wd1[...])
    hw_ref[...] = _dot(xb, w1) * dinv
    dinv_ref[...] = dinv
    g1w = jnp.where(is_p, ga1w[...], gb1w[...])
    g1b = jnp.where(is_p, ga1b[...], gb1b[...])
    g2w = jnp.where(is_p, ga2w[...], gb2w[...])
    g2b = jnp.where(is_p, ga2b[...], gb2b[...])
    t = jnp.maximum(_dot(xb, g1w) + g1b, 0.0)
    g = _dot(t, g2w) + g2b
    g_ref[...] = g
    bm = jnp.max(g, keepdims=True)

    @pl.when(i == 0)
    def _():
        gmp_ref[...] = bm
        gmd_ref[...] = bm - 1.0

    @pl.when((i > 0) & is_p)
    def _():
        gmp_ref[...] = jnp.maximum(gmp_ref[...], bm)

    @pl.when(i == NBP)
    def _():
        gmd_ref[...] = bm

    @pl.when(i > NBP)
    def _():
        gmd_ref[...] = jnp.maximum(gmd_ref[...], bm)


def _tc_k1(x, degacc, p):
    return pl.pallas_call(
        _k1_body,
        grid=(NBLK,),
        in_specs=[
            pl.BlockSpec((RB, D), lambda i: (i, 0)),
            pl.BlockSpec((RB, LANES), lambda i: (i, 0)),
        ] + [pl.BlockSpec(w.shape, lambda i: (0, 0))
             for w in (p["Wp1"], p["Wd1"], p["Ga1"], p["Gb1"],
                       p["ga1r"], p["gb1r"], p["Ga2"], p["Gb2"],
                       p["ga2r"], p["gb2r"])],
        out_specs=[
            pl.BlockSpec((RB, 32), lambda i: (i, 0)),
            pl.BlockSpec((RB, 1), lambda i: (i, 0)),
            pl.BlockSpec((RB, 1), lambda i: (i, 0)),
            pl.BlockSpec((1, 1), lambda i: (0, 0)),
            pl.BlockSpec((1, 1), lambda i: (0, 0)),
        ],
        out_shape=[
            jax.ShapeDtypeStruct((M, 32), F32),
            jax.ShapeDtypeStruct((M, 1), F32),
            jax.ShapeDtypeStruct((M, 1), F32),
            jax.ShapeDtypeStruct((1, 1), F32),
            jax.ShapeDtypeStruct((1, 1), F32),
        ],
    )(x, degacc, p["Wp1"], p["Wd1"], p["Ga1"], p["Gb1"], p["ga1r"],
      p["gb1r"], p["Ga2"], p["Gb2"], p["ga2r"], p["gb2r"])


def _pool_body(h3_ref, g_ref, gmp_ref, gmd_ref, x_ref, batch_ref,
               l1w_ref, l1b_ref, l2w_ref, l2b_ref,
               shp, sdp, scp, sxp, shd, sdd, scd, sxd, out_ref):
    i = pl.program_id(0)
    is_p = i < NBP
    h3 = h3_ref[...]
    gm = jnp.where(is_p, gmp_ref[...], gmd_ref[...])
    ge = jnp.exp(g_ref[...] - gm)
    iota = lax.broadcasted_iota(jnp.int32, (1, B), 1).astype(F32)
    oh = (batch_ref[...] == iota).astype(F32)
    sh = _dot_t(oh, h3)
    sden = _dot_t(oh, ge)
    scnt = _dot_t(oh, jnp.ones((RB, 1), F32))
    sx = _dot_t(oh, ge * x_ref[...])

    @pl.when(i == 0)
    def _():
        shp[...] = sh
        sdp[...] = sden
        scp[...] = scnt
        sxp[...] = sx
        out_ref[...] = jnp.zeros((B, 1), F32)

    @pl.when((i > 0) & is_p)
    def _():
        shp[...] += sh
        sdp[...] += sden
        scp[...] += scnt
        sxp[...] += sx

    @pl.when(i == NBP)
    def _():
        shd[...] = sh
        sdd[...] = sden
        scd[...] = scnt
        sxd[...] = sx

    @pl.when(i > NBP)
    def _():
        shd[...] += sh
        sdd[...] += sden
        scd[...] += scnt
        sxd[...] += sx

    @pl.when(i == NBLK - 1)
    def _():
        pp = shp[...] / jnp.maximum(scp[...], 1.0)
        pd = shd[...] / jnp.maximum(scd[...], 1.0)
        ap = sxp[...] / jnp.maximum(sdp[...], 1e-12)
        ad = sxd[...] / jnp.maximum(sdd[...], 1e-12)
        z = (_dot(pp, l1w_ref[0:16]) + _dot(pd, l1w_ref[16:32])
             + _dot(ap, l1w_ref[32:160]) + _dot(ad, l1w_ref[160:288])
             + l1b_ref[...])
        z = jnp.maximum(z, 0.0)
        out_ref[...] = _dot(z, l2w_ref[...]) + l2b_ref[...]


def _tc_pool(h3, g, gmp, gmd, x, batch, l1w, l1b, l2w, l2b):
    zspec = pl.BlockSpec((1, 1), lambda i: (0, 0))
    outs = [
        pl.BlockSpec((B, 16), lambda i: (0, 0)),
        pl.BlockSpec((B, 1), lambda i: (0, 0)),
        pl.BlockSpec((B, 1), lambda i: (0, 0)),
        pl.BlockSpec((B, D), lambda i: (0, 0)),
    ]
    shapes = [
        jax.ShapeDtypeStruct((B, 16), F32),
        jax.ShapeDtypeStruct((B, 1), F32),
        jax.ShapeDtypeStruct((B, 1), F32),
        jax.ShapeDtypeStruct((B, D), F32),
    ]
    res = pl.pallas_call(
        _pool_body,
        grid=(NBLK,),
        in_specs=[
            pl.BlockSpec((RB, 16), lambda i: (i, 0)),
            pl.BlockSpec((RB, 1), lambda i: (i, 0)),
            zspec,
            zspec,
            pl.BlockSpec((RB, D), lambda i: (i, 0)),
            pl.BlockSpec((RB, 1), lambda i: (i, 0)),
            pl.BlockSpec((288, 16), lambda i: (0, 0)),
            pl.BlockSpec((1, 16), lambda i: (0, 0)),
            pl.BlockSpec((16, 1), lambda i: (0, 0)),
            pl.BlockSpec((1, 1), lambda i: (0, 0)),
        ],
        out_specs=outs + outs + [pl.BlockSpec((B, 1), lambda i: (0, 0))],
        out_shape=shapes + shapes + [jax.ShapeDtypeStruct((B, 1), F32)],
    )(h3, g, gmp, gmd, x, batch, l1w, l1b, l2w, l2b)
    return res[-1]


def kernel(x_p, x_d, edge_attr_p, edge_attr_d, edge_index_p, edge_index_d,
           batch_p, batch_d, params):
    del edge_attr_p, edge_attr_d
    p = dict(params)
    for k in ("ga1", "gb1", "ga2", "gb2", "bp1", "bp2", "bp3",
              "bd1", "bd2", "bd3", "l1", "l2"):
        p[k + "r"] = p[k].reshape(1, -1).astype(F32)

    pad_n = NP_PAD - N
    zrows = jnp.zeros((pad_n, D), F32)
    x = jnp.concatenate([x_p, zrows, x_d, zrows], axis=0)

    ar = jnp.arange(NP_PAD, dtype=jnp.int32)
    epad_n = EEPG - (E + NP_PAD)

    def edges_for(g, ei):
        srcg = jnp.concatenate([
            ei[0].astype(jnp.int32), ar,
            jnp.full((epad_n,), PAD_ROW, jnp.int32)]) + g * NP_PAD
        dstl = jnp.concatenate([
            ei[1].astype(jnp.int32), ar,
            jnp.full((epad_n,), PAD_ROW, jnp.int32)])
        return srcg.reshape(NS, CH, K), dstl.reshape(NS, CH, K)

    sp, dp_ = edges_for(0, edge_index_p)
    sd, dd_ = edges_for(1, edge_index_d)
    src4 = jnp.stack([sp, sd])
    dst4 = jnp.stack([dp_, dd_])

    bpad = jnp.full((pad_n,), B, batch_p.dtype)
    batch = jnp.concatenate([batch_p, bpad, batch_d, bpad])
    batch = batch.astype(F32).reshape(M, 1)

    deg = _get_sc_degree()(dst4)
    z1, dinv, g, gmp, gmd = _tc_k1(x, deg, p)
    w2 = jnp.stack([p["Wp2"], p["Wd2"]])
    w3 = jnp.stack([p["Wp3"], p["Wd3"]])
    h3, _, _ = _get_sc_layers()(
        z1, src4, dst4, dinv.reshape(M),
        w2, w3,
        jnp.stack([p["bp1"].reshape(2, LANES),
                   p["bd1"].reshape(2, LANES)]).astype(F32),
        jnp.stack([p["bp2r"], p["bd2r"]]),
        jnp.stack([p["bp3r"], p["bd3r"]]))
    return _tc_pool(h3, g, gmp, gmd, x, batch,
                    p["L1"], p["l1r"], p["L2"], p["l2r"])
